# baseline (speedup 1.0000x reference)
import functools

import jax
import jax.numpy as jnp
from jax import lax
from jax.experimental import pallas as pl
from jax.experimental.pallas import tpu as pltpu
from jax.experimental.pallas import tpu_sc as plsc

_VT = 8192
_K = 524288


def _transpose_body(x1_ref, x2_ref, eye_ref, out_ref):
  dn = (((0,), (0,)), ((), ()))
  t1 = jax.lax.dot_general(x1_ref[...], eye_ref[...], dn,
                           preferred_element_type=jnp.float32)
  t2 = jax.lax.dot_general(x2_ref[...], eye_ref[...], dn,
                           preferred_element_type=jnp.float32)
  out_ref[...] = jnp.concatenate([t1, t2], axis=1)


def _tc_transpose_pack(embT):
  D, V = embT.shape
  eye = jnp.eye(D, dtype=jnp.float32)
  hi_blocks = _K // _VT
  last_block = pl.cdiv(V, _VT) - 1

  def hi_map(i):
    return (0, jnp.minimum(i + hi_blocks, last_block))

  return pl.pallas_call(
      _transpose_body,
      grid=(_K // _VT,),
      in_specs=[
          pl.BlockSpec((D, _VT), lambda i: (0, i)),
          pl.BlockSpec((D, _VT), hi_map),
          pl.BlockSpec((D, D), lambda i: (0, 0)),
      ],
      out_specs=pl.BlockSpec((_VT, 128), lambda i: (i, 0)),
      out_shape=jax.ShapeDtypeStruct((_K, 128), jnp.float32),
  )(embT, embT, eye)


def _sc_gather(table, idx):
  V, D = table.shape
  B = idx.shape[0]
  info = plsc.get_sparse_core_info()
  NC, NS = info.num_cores, info.num_subcores
  NW = NC * NS
  b_per_w = B // NW
  idx3 = idx.reshape(NW, 1, b_per_w)

  mesh = plsc.VectorSubcoreMesh(core_axis_name="c", subcore_axis_name="s")

  @functools.partial(
      pl.kernel,
      mesh=mesh,
      out_type=jax.ShapeDtypeStruct((B, D), jnp.float32),
      scratch_types=[
          pltpu.VMEM((1, b_per_w), jnp.int32),
          pltpu.VMEM((b_per_w, D), jnp.float32),
          pltpu.SemaphoreType.DMA,
      ],
  )
  def gather_kernel(table_hbm, idx_hbm, out_hbm, idx_v, rows_v, sem):
    wid = lax.axis_index("s") * NC + lax.axis_index("c")
    base = wid * b_per_w
    pltpu.sync_copy(idx_hbm.at[wid], idx_v)
    pltpu.async_copy(table_hbm.at[idx_v.at[0]], rows_v, sem).wait()
    pltpu.sync_copy(rows_v, out_hbm.at[pl.ds(base, b_per_w)])

  return gather_kernel(table, idx3)


def _mlp_body(x2_ref, sel_ref, w1_ref, b1_ref, w2_ref, b2_ref, w3t_ref,
              b3_ref, out_ref):
  x2 = x2_ref[...]
  lo = x2[:, :64]
  hi = x2[:, 64:]
  x = jnp.where(sel_ref[...] > 0, lo, hi)
  h = jnp.dot(x, w1_ref[...], preferred_element_type=jnp.float32)
  h = jnp.maximum(h + b1_ref[...], 0.0)
  h = jnp.dot(h, w2_ref[...], preferred_element_type=jnp.float32)
  h = jnp.maximum(h + b2_ref[...], 0.0)
  ot = jax.lax.dot_general(
      w3t_ref[...], h, (((1,), (1,)), ((), ())),
      preferred_element_type=jnp.float32,
  )
  out_ref[...] = ot + b3_ref[...]


def _tc_mlp(x2, sel, W1, b1, W2, b2, W3T, b3, tb=2048):
  B = x2.shape[0]
  D, H = W1.shape
  N = W3T.shape[0]
  b1r = b1.reshape(1, H)
  b2r = b2.reshape(1, H)
  b3c = b3.reshape(N, 1)
  return pl.pallas_call(
      _mlp_body,
      grid=(B // tb,),
      in_specs=[
          pl.BlockSpec((tb, 128), lambda i: (i, 0)),
          pl.BlockSpec((tb, 1), lambda i: (i, 0)),
          pl.BlockSpec((D, H), lambda i: (0, 0)),
          pl.BlockSpec((1, H), lambda i: (0, 0)),
          pl.BlockSpec((H, H), lambda i: (0, 0)),
          pl.BlockSpec((1, H), lambda i: (0, 0)),
          pl.BlockSpec((N, H), lambda i: (0, 0)),
          pl.BlockSpec((N, 1), lambda i: (0, 0)),
      ],
      out_specs=pl.BlockSpec((N, tb), lambda i: (0, i)),
      out_shape=jax.ShapeDtypeStruct((N, B), jnp.float32),
  )(x2, sel, W1, b1r, W2, b2r, W3T, b3c)


def kernel(text, offsets, emb, W1, b1, W2, b2, W3, b3):
  del offsets
  P = _tc_transpose_pack(emb.T)
  in_lo = text < _K
  idx = jnp.where(in_lo, text, text - _K)
  sel = in_lo.astype(jnp.float32).reshape(-1, 1)
  x2 = _sc_gather(P, idx)
  outT = _tc_mlp(x2, sel, W1, b1, W2, b2, W3.T, b3)
  return outT.T


# speedup vs baseline: 1.0510x; 1.0510x over previous
import functools

import jax
import jax.numpy as jnp
from jax import lax
from jax.experimental import pallas as pl
from jax.experimental.pallas import tpu as pltpu
from jax.experimental.pallas import tpu_sc as plsc

_VT = 16384
_K = 524288


def _transpose_body(x1_ref, x2_ref, eye_ref, out_ref):
  dn = (((0,), (0,)), ((), ()))
  t1 = jax.lax.dot_general(x1_ref[...], eye_ref[...], dn,
                           preferred_element_type=jnp.float32)
  t2 = jax.lax.dot_general(x2_ref[...], eye_ref[...], dn,
                           preferred_element_type=jnp.float32)
  out_ref[...] = jnp.concatenate([t1, t2], axis=1)


def _tc_transpose_pack(embT):
  D, V = embT.shape
  eye = jnp.eye(D, dtype=jnp.float32)
  hi_blocks = _K // _VT
  last_block = pl.cdiv(V, _VT) - 1

  def hi_map(i):
    return (0, jnp.minimum(i + hi_blocks, last_block))

  return pl.pallas_call(
      _transpose_body,
      grid=(_K // _VT,),
      in_specs=[
          pl.BlockSpec((D, _VT), lambda i: (0, i)),
          pl.BlockSpec((D, _VT), hi_map),
          pl.BlockSpec((D, D), lambda i: (0, 0)),
      ],
      out_specs=pl.BlockSpec((_VT, 128), lambda i: (i, 0)),
      out_shape=jax.ShapeDtypeStruct((_K, 128), jnp.float32),
  )(embT, embT, eye)


def _sc_gather(table, idx):
  V, D = table.shape
  B = idx.shape[0]
  info = plsc.get_sparse_core_info()
  NC, NS = info.num_cores, info.num_subcores
  NW = NC * NS
  b_per_w = B // NW
  idx3 = idx.reshape(NW, 1, b_per_w)

  mesh = plsc.VectorSubcoreMesh(core_axis_name="c", subcore_axis_name="s")

  @functools.partial(
      pl.kernel,
      mesh=mesh,
      out_type=jax.ShapeDtypeStruct((B, D), jnp.float32),
      scratch_types=[
          pltpu.VMEM((1, b_per_w), jnp.int32),
          pltpu.VMEM((b_per_w, D), jnp.float32),
          pltpu.SemaphoreType.DMA,
      ],
  )
  def gather_kernel(table_hbm, idx_hbm, out_hbm, idx_v, rows_v, sem):
    wid = lax.axis_index("s") * NC + lax.axis_index("c")
    base = wid * b_per_w
    pltpu.sync_copy(idx_hbm.at[wid], idx_v)
    pltpu.async_copy(table_hbm.at[idx_v.at[0]], rows_v, sem).wait()
    pltpu.sync_copy(rows_v, out_hbm.at[pl.ds(base, b_per_w)])

  return gather_kernel(table, idx3)


def _mlp_body(x2_ref, sel_ref, w1_ref, b1_ref, w2_ref, b2_ref, w3t_ref,
              b3_ref, out_ref):
  x2 = x2_ref[...]
  lo = x2[:, :64]
  hi = x2[:, 64:]
  x = jnp.where(sel_ref[...] > 0, lo, hi)
  h = jnp.dot(x, w1_ref[...], preferred_element_type=jnp.float32)
  h = jnp.maximum(h + b1_ref[...], 0.0)
  h = jnp.dot(h, w2_ref[...], preferred_element_type=jnp.float32)
  h = jnp.maximum(h + b2_ref[...], 0.0)
  ot = jax.lax.dot_general(
      w3t_ref[...], h, (((1,), (1,)), ((), ())),
      preferred_element_type=jnp.float32,
  )
  out_ref[...] = ot + b3_ref[...]


def _tc_mlp(x2, sel, W1, b1, W2, b2, W3T, b3, tb=2048):
  B = x2.shape[0]
  D, H = W1.shape
  N = W3T.shape[0]
  b1r = b1.reshape(1, H)
  b2r = b2.reshape(1, H)
  b3c = b3.reshape(N, 1)
  return pl.pallas_call(
      _mlp_body,
      grid=(B // tb,),
      in_specs=[
          pl.BlockSpec((tb, 128), lambda i: (i, 0)),
          pl.BlockSpec((tb, 1), lambda i: (i, 0)),
          pl.BlockSpec((D, H), lambda i: (0, 0)),
          pl.BlockSpec((1, H), lambda i: (0, 0)),
          pl.BlockSpec((H, H), lambda i: (0, 0)),
          pl.BlockSpec((1, H), lambda i: (0, 0)),
          pl.BlockSpec((N, H), lambda i: (0, 0)),
          pl.BlockSpec((N, 1), lambda i: (0, 0)),
      ],
      out_specs=pl.BlockSpec((N, tb), lambda i: (0, i)),
      out_shape=jax.ShapeDtypeStruct((N, B), jnp.float32),
  )(x2, sel, W1, b1r, W2, b2r, W3T, b3c)


def kernel(text, offsets, emb, W1, b1, W2, b2, W3, b3):
  del offsets
  P = _tc_transpose_pack(emb.T)
  in_lo = text < _K
  idx = jnp.where(in_lo, text, text - _K)
  sel = in_lo.astype(jnp.float32).reshape(-1, 1)
  x2 = _sc_gather(P, idx)
  outT = _tc_mlp(x2, sel, W1, b1, W2, b2, W3.T, b3)
  return outT.T
